# baseline (speedup 1.0000x reference)
import functools

import jax
import jax.numpy as jnp
from jax import lax
from jax.experimental import pallas as pl
from jax.experimental.pallas import tpu as pltpu
from jax.experimental.pallas import tpu_sc as plsc

VOCAB = 100000
D = 64

NC = 2
NS = 16
NW = NC * NS

C = 128
K = 4
ROWS_G = C * K

NBUF = 3
H = 4


def _emb_body(nchunks, ngroups, x_ref, tab_ref, out_ref, idx_v, rows_v,
              sem_g0, sem_g1, sem_g2, sem_o0, sem_o1, sem_o2):
    sem_g = (sem_g0, sem_g1, sem_g2)
    sem_o = (sem_o0, sem_o1, sem_o2)
    wid = lax.axis_index("s") * NC + lax.axis_index("c")
    npw = nchunks * C
    base = wid * npw

    pltpu.sync_copy(x_ref.at[wid], idx_v)

    def fire_gathers(g, b):
        for j in range(K):
            pltpu.async_copy(tab_ref.at[idx_v.at[g * K + j]],
                             rows_v.at[b].at[pl.ds(j * C, C)], sem_g[b])

    def drain(buf, sem):
        pltpu.make_async_copy(out_ref.at[pl.ds(base, ROWS_G)], buf, sem).wait()

    for b in range(NBUF):
        fire_gathers(b, b)

    nsteps = ngroups // NBUF

    def step(gp, _):
        for b in range(NBUF):
            g = gp * NBUF + b
            drain(rows_v.at[b], sem_g[b])
            pltpu.async_copy(rows_v.at[b],
                             out_ref.at[pl.ds(base + g * ROWS_G, ROWS_G)],
                             sem_o[b])

            @pl.when(gp < nsteps - 1)
            def _():
                drain(rows_v.at[b], sem_o[b])
                fire_gathers(g + NBUF, b)
        return 0

    lax.fori_loop(0, nsteps, step, 0)
    for b in range(NBUF):
        drain(rows_v.at[b], sem_o[b])


def _sc_gather(xw_h, tab, nrows):
    nchunks = nrows // (NW * C)
    ngroups = nchunks // K
    mesh = plsc.VectorSubcoreMesh(core_axis_name="c", subcore_axis_name="s")
    k = pl.kernel(
        functools.partial(_emb_body, nchunks, ngroups),
        out_type=jax.ShapeDtypeStruct((nrows, D), jnp.float32),
        mesh=mesh,
        compiler_params=pltpu.CompilerParams(use_tc_tiling_on_sc=False),
        scratch_types=[
            pltpu.VMEM((nchunks, C), jnp.int32),
            pltpu.VMEM((NBUF, ROWS_G, D), jnp.float32),
            pltpu.SemaphoreType.DMA,
            pltpu.SemaphoreType.DMA,
            pltpu.SemaphoreType.DMA,
            pltpu.SemaphoreType.DMA,
            pltpu.SemaphoreType.DMA,
            pltpu.SemaphoreType.DMA,
        ],
    )
    return k(xw_h, tab)


def _tpose_body(eye_ref, i_ref, o_ref):
    eye = eye_ref[...]
    xb = i_ref[...]
    for t24 in range(24):
        iloc, jp = t24 // 12, t24 % 12
        v = xb[:, t24, :]
        t = lax.dot_general(eye, v, (((1,), (1,)), ((), ())),
                            preferred_element_type=jnp.float32,
                            precision=lax.Precision.DEFAULT)
        o_ref[iloc, 2 * jp, :, 0, :, :] = t[0:64].reshape(8, 8, 128)
        o_ref[iloc, 2 * jp + 1, :, 0, :, :] = t[64:128].reshape(8, 8, 128)


def _tc_chunk(eye, i3_h, t6_prev, bt0, nbt):
    in_specs = [pl.BlockSpec((128, 128), lambda g, bt: (0, 0)),
                pl.BlockSpec((128, 24, 128), lambda g, bt: (bt, g, 0))]
    args = [eye, i3_h]
    aliases = {}
    body = _tpose_body
    if t6_prev is not None:
        in_specs.append(pl.BlockSpec(memory_space=pl.ANY))
        args.append(t6_prev)
        aliases = {2: 0}
        body = lambda eye_ref, i_ref, _, o_ref: _tpose_body(eye_ref, i_ref, o_ref)
    return pl.pallas_call(
        body,
        grid=(12, nbt),
        in_specs=in_specs,
        out_specs=pl.BlockSpec((2, 24, 8, 1, 8, 128),
                               lambda g, bt: (g, 0, 0, bt0 + bt, 0, 0)),
        out_shape=jax.ShapeDtypeStruct((24, 24, 8, 8, 8, 128), jnp.float32),
        input_output_aliases=aliases,
    )(*args)


def _table_prep_body(s0_ref, s1_ref, t_ref, o_ref):
    w = t_ref[...]
    r0 = lax.dot_general(s0_ref[...], w, (((1,), (1,)), ((), ())),
                         preferred_element_type=jnp.float32)
    r1 = lax.dot_general(s1_ref[...], w, (((1,), (1,)), ((), ())),
                         preferred_element_type=jnp.float32)
    o_ref[...] = jnp.concatenate([r0, r1], axis=1)


def _table_prep(tok_emb_code):
    tt = tok_emb_code.T
    s = jnp.arange(256, dtype=jnp.int32)[:, None]
    k = jnp.arange(512, dtype=jnp.int32)[None, :]
    s0 = (k == 2 * s).astype(jnp.float32)
    s1 = (k == 2 * s + 1).astype(jnp.float32)
    out = pl.pallas_call(
        _table_prep_body,
        grid=(196,),
        in_specs=[pl.BlockSpec((256, 512), lambda v: (0, 0)),
                  pl.BlockSpec((256, 512), lambda v: (0, 0)),
                  pl.BlockSpec((64, 512), lambda v: (0, v))],
        out_specs=pl.BlockSpec((256, 128), lambda v: (v, 0)),
        out_shape=jax.ShapeDtypeStruct((50000, 128), jnp.float32),
    )(s0, s1, tt)
    return out.reshape(VOCAB, D)


def kernel(x, tok_emb_code):
    orig_shape = x.shape
    n = x.size
    assert orig_shape == (1024, 24, 24)
    rows_h = n // H
    bt_h = 8 // H

    xw = x.reshape(H, NW, rows_h // (NW * C), C).astype(jnp.int32)
    tab = _table_prep(tok_emb_code)
    eye = jnp.eye(128, dtype=jnp.float32)
    t6 = None
    for h in range(H):
        flat_h = _sc_gather(xw[h], tab, rows_h)
        i3_h = flat_h.reshape(rows_h // 576, 288, 128)
        t6 = _tc_chunk(eye, i3_h, t6, h * bt_h, bt_h)
    return t6.transpose(3, 5, 0, 1, 2, 4).reshape(1024, 24, 24, D)


# speedup vs baseline: 1.2425x; 1.2425x over previous
import functools

import jax
import jax.numpy as jnp
from jax import lax
from jax.experimental import pallas as pl
from jax.experimental.pallas import tpu as pltpu
from jax.experimental.pallas import tpu_sc as plsc

VOCAB = 100000
D = 64

NC = 2
NS = 16
NW = NC * NS

C = 128
K = 4
ROWS_G = C * K

NBUF = 3
H = 4


def _emb_body(nchunks, ngroups, x_ref, tab_ref, out_ref, idx_v, rows_v,
              sem_g0, sem_g1, sem_g2, sem_o0, sem_o1, sem_o2):
    sem_g = (sem_g0, sem_g1, sem_g2)
    sem_o = (sem_o0, sem_o1, sem_o2)
    wid = lax.axis_index("s") * NC + lax.axis_index("c")
    npw = nchunks * C
    base = wid * npw

    pltpu.sync_copy(x_ref.at[wid], idx_v)

    def fire_gathers(g, b):
        for j in range(K):
            pltpu.async_copy(tab_ref.at[idx_v.at[g * K + j]],
                             rows_v.at[b].at[pl.ds(j * C, C)], sem_g[b])

    def drain(buf, sem):
        pltpu.make_async_copy(out_ref.at[pl.ds(base, ROWS_G)], buf, sem).wait()

    for b in range(NBUF):
        fire_gathers(b, b)

    nsteps = ngroups // NBUF

    def step(gp, _):
        for b in range(NBUF):
            g = gp * NBUF + b
            drain(rows_v.at[b], sem_g[b])
            pltpu.async_copy(rows_v.at[b],
                             out_ref.at[pl.ds(base + g * ROWS_G, ROWS_G)],
                             sem_o[b])

            @pl.when(gp < nsteps - 1)
            def _():
                drain(rows_v.at[b], sem_o[b])
                fire_gathers(g + NBUF, b)
        return 0

    lax.fori_loop(0, nsteps, step, 0)
    for b in range(NBUF):
        drain(rows_v.at[b], sem_o[b])


def _sc_gather(xw_h, tab, nrows):
    nchunks = nrows // (NW * C)
    ngroups = nchunks // K
    mesh = plsc.VectorSubcoreMesh(core_axis_name="c", subcore_axis_name="s")
    k = pl.kernel(
        functools.partial(_emb_body, nchunks, ngroups),
        out_type=jax.ShapeDtypeStruct((nrows, D), jnp.float32),
        mesh=mesh,
        compiler_params=pltpu.CompilerParams(use_tc_tiling_on_sc=False),
        scratch_types=[
            pltpu.VMEM((nchunks, C), jnp.int32),
            pltpu.VMEM((NBUF, ROWS_G, D), jnp.float32),
            pltpu.SemaphoreType.DMA,
            pltpu.SemaphoreType.DMA,
            pltpu.SemaphoreType.DMA,
            pltpu.SemaphoreType.DMA,
            pltpu.SemaphoreType.DMA,
            pltpu.SemaphoreType.DMA,
        ],
    )
    return k(xw_h, tab)


def _tpose_body(eye_ref, i_ref, o_ref):
    eye = eye_ref[...]
    xb = i_ref[...]
    for t24 in range(24):
        iloc, jp = t24 // 12, t24 % 12
        v = xb[:, t24, :]
        t = lax.dot_general(eye, v, (((1,), (1,)), ((), ())),
                            preferred_element_type=jnp.float32,
                            precision=lax.Precision.DEFAULT)
        o_ref[iloc, 2 * jp, :, 0, :, :] = t[0:64].reshape(8, 8, 128)
        o_ref[iloc, 2 * jp + 1, :, 0, :, :] = t[64:128].reshape(8, 8, 128)


def _tc_chunk(eye, i3_h, t6_prev, bt0, nbt):
    in_specs = [pl.BlockSpec((128, 128), lambda g, bt: (0, 0)),
                pl.BlockSpec((128, 24, 128), lambda g, bt: (bt, g, 0))]
    args = [eye, i3_h]
    aliases = {}
    body = _tpose_body
    if t6_prev is not None:
        in_specs.append(pl.BlockSpec(memory_space=pl.ANY))
        args.append(t6_prev)
        aliases = {2: 0}
        body = lambda eye_ref, i_ref, _, o_ref: _tpose_body(eye_ref, i_ref, o_ref)
    return pl.pallas_call(
        body,
        grid=(12, nbt),
        in_specs=in_specs,
        out_specs=pl.BlockSpec((2, 24, 8, 1, 8, 128),
                               lambda g, bt: (g, 0, 0, bt0 + bt, 0, 0)),
        out_shape=jax.ShapeDtypeStruct((24, 24, 8, 8, 8, 128), jnp.float32),
        input_output_aliases=aliases,
    )(*args)


def kernel(x, tok_emb_code):
    orig_shape = x.shape
    n = x.size
    assert orig_shape == (1024, 24, 24)
    rows_h = n // H
    bt_h = 8 // H

    xw = x.reshape(H, NW, rows_h // (NW * C), C).astype(jnp.int32)
    eye = jnp.eye(128, dtype=jnp.float32)
    t6 = None
    for h in range(H):
        flat_h = _sc_gather(xw[h], tok_emb_code, rows_h)
        i3_h = flat_h.reshape(rows_h // 576, 288, 128)
        t6 = _tc_chunk(eye, i3_h, t6, h * bt_h, bt_h)
    return t6.transpose(3, 5, 0, 1, 2, 4).reshape(1024, 24, 24, D)
